# Initial kernel scaffold; baseline (speedup 1.0000x reference)
import jax
import jax.numpy as jnp
from jax.experimental import pallas as pl


def kernel(toks, table):
    raise NotImplementedError("write your pallas kernel here")


# speedup vs baseline: 1.2429x; 1.2429x over previous
import functools

import jax
import jax.numpy as jnp
from jax import lax
from jax.experimental import pallas as pl
from jax.experimental.pallas import tpu as pltpu
from jax.experimental.pallas import tpu_sc as plsc

VOCAB_DIM = 768
B_TOK = 1024
L_TOK = 50
N_ROWS = B_TOK * L_TOK

_info = plsc.get_sparse_core_info()
NC = _info.num_cores
NS = _info.num_subcores
NW = NC * NS
B_PER_W = N_ROWS // NW

CHUNK = 80
N_CHUNKS = B_PER_W // CHUNK


def _sc_gather(toks_grouped, table):
    mesh = plsc.VectorSubcoreMesh(core_axis_name="c", subcore_axis_name="s")

    @functools.partial(
        pl.kernel,
        mesh=mesh,
        out_type=jax.ShapeDtypeStruct((N_ROWS, VOCAB_DIM), jnp.float32),
        scratch_types=[
            pltpu.VMEM((N_CHUNKS, CHUNK), jnp.int32),
            pltpu.VMEM((CHUNK, VOCAB_DIM), jnp.float32),
            pltpu.SemaphoreType.DMA,
        ],
    )
    def k(toks_hbm, table_hbm, out_hbm, idx_v, rows_v, sem):
        wid = lax.axis_index("s") * NC + lax.axis_index("c")
        base = wid * B_PER_W
        pltpu.sync_copy(toks_hbm.at[wid], idx_v)
        for c in range(N_CHUNKS):
            pltpu.async_copy(table_hbm.at[idx_v.at[c]], rows_v, sem).wait()
            pltpu.sync_copy(rows_v, out_hbm.at[pl.ds(base + c * CHUNK, CHUNK)])

    return k(toks_grouped, table)


@jax.jit
def kernel(toks, table):
    toks_grouped = toks.reshape(NW, N_CHUNKS, CHUNK)
    flat = _sc_gather(toks_grouped, table)
    return flat.reshape(B_TOK, L_TOK, VOCAB_DIM)


# speedup vs baseline: 1.2735x; 1.0247x over previous
import functools

import jax
import jax.numpy as jnp
from jax import lax
from jax.experimental import pallas as pl
from jax.experimental.pallas import tpu as pltpu
from jax.experimental.pallas import tpu_sc as plsc

VOCAB_DIM = 768
B_TOK = 1024
L_TOK = 50
N_ROWS = B_TOK * L_TOK

_info = plsc.get_sparse_core_info()
NC = _info.num_cores
NS = _info.num_subcores
NW = NC * NS
B_PER_W = N_ROWS // NW

CHUNK = 40
N_CHUNKS = B_PER_W // CHUNK
NBUF = 4


def _sc_gather(toks_grouped, table):
    mesh = plsc.VectorSubcoreMesh(core_axis_name="c", subcore_axis_name="s")

    @functools.partial(
        pl.kernel,
        mesh=mesh,
        out_type=jax.ShapeDtypeStruct((N_ROWS, VOCAB_DIM), jnp.float32),
        scratch_types=[
            pltpu.VMEM((N_CHUNKS, CHUNK), jnp.int32),
        ]
        + [pltpu.VMEM((CHUNK, VOCAB_DIM), jnp.float32) for _ in range(NBUF)]
        + [pltpu.SemaphoreType.DMA for _ in range(2 * NBUF)],
    )
    def k(toks_hbm, table_hbm, out_hbm, idx_v, *scratch):
        bufs = scratch[:NBUF]
        gsem = scratch[NBUF : 2 * NBUF]
        wsem = scratch[2 * NBUF :]
        wid = lax.axis_index("s") * NC + lax.axis_index("c")
        base = wid * B_PER_W
        pltpu.sync_copy(toks_hbm.at[wid], idx_v)

        def gather(j):
            q = j % NBUF
            return pltpu.async_copy(table_hbm.at[idx_v.at[j]], bufs[q], gsem[q])

        def writeback(c):
            p = c % NBUF
            return pltpu.async_copy(
                bufs[p], out_hbm.at[pl.ds(base + c * CHUNK, CHUNK)], wsem[p]
            )

        gh = {}
        wh = {}
        gh[0] = gather(0)
        gh[1] = gather(1)
        for c in range(N_CHUNKS):
            j = c + 2
            if j < N_CHUNKS:
                if j >= NBUF:
                    wh[j - NBUF].wait()
                gh[j] = gather(j)
            gh[c].wait()
            wh[c] = writeback(c)
        for c in range(N_CHUNKS - NBUF, N_CHUNKS):
            wh[c].wait()

    return k(toks_grouped, table)


@jax.jit
def kernel(toks, table):
    toks_grouped = toks.reshape(NW, N_CHUNKS, CHUNK)
    flat = _sc_gather(toks_grouped, table)
    return flat.reshape(B_TOK, L_TOK, VOCAB_DIM)


# speedup vs baseline: 3.7382x; 2.9353x over previous
import functools

import jax
import jax.numpy as jnp
from jax import lax
from jax.experimental import pallas as pl
from jax.experimental.pallas import tpu as pltpu
from jax.experimental.pallas import tpu_sc as plsc

VOCAB_DIM = 768
B_TOK = 1024
L_TOK = 50
N_ROWS = B_TOK * L_TOK

_info = plsc.get_sparse_core_info()
NC = _info.num_cores
NS = _info.num_subcores
NW = NC * NS
B_PER_W = N_ROWS // NW

CHUNK = 40
N_CHUNKS = B_PER_W // CHUNK
NBUF = 4


def _sc_gather(toks_grouped, table):
    mesh = plsc.VectorSubcoreMesh(core_axis_name="c", subcore_axis_name="s")

    @functools.partial(
        pl.kernel,
        mesh=mesh,
        out_type=jax.ShapeDtypeStruct((N_ROWS, VOCAB_DIM), jnp.float32),
        scratch_types=[
            pltpu.VMEM((N_CHUNKS, CHUNK), jnp.int32),
        ]
        + [pltpu.VMEM((CHUNK, VOCAB_DIM), jnp.float32) for _ in range(NBUF)]
        + [pltpu.SemaphoreType.DMA for _ in range(2 * NBUF)],
    )
    def k(toks_hbm, table_hbm, out_hbm, idx_v, *scratch):
        bufs = scratch[:NBUF]
        gsem = scratch[NBUF : 2 * NBUF]
        wsem = scratch[2 * NBUF :]
        wid = lax.axis_index("s") * NC + lax.axis_index("c")
        base = wid * B_PER_W
        pltpu.sync_copy(toks_hbm.at[wid], idx_v)

        def gather(j):
            q = j % NBUF
            return pltpu.async_copy(table_hbm.at[idx_v.at[j]], bufs[q], gsem[q])

        def writeback(c):
            p = c % NBUF
            return pltpu.async_copy(
                bufs[p], out_hbm.at[pl.ds(base + c * CHUNK, CHUNK)], wsem[p]
            )

        gh = {}
        wh = {}
        gh[0] = gather(0)
        gh[1] = gather(1)
        for c in range(N_CHUNKS):
            j = c + 2
            if j < N_CHUNKS:
                if j >= NBUF:
                    wh[j - NBUF].wait()
                gh[j] = gather(j)
            gh[c].wait()
            wh[c] = writeback(c)
        for c in range(N_CHUNKS - NBUF, N_CHUNKS):
            wh[c].wait()

    return k(toks_grouped, table)


@jax.jit
def kernel(toks, table):
    toks_grouped = toks.T.reshape(NW, N_CHUNKS, CHUNK)
    flat = _sc_gather(toks_grouped, table)
    return flat.reshape(L_TOK, B_TOK, VOCAB_DIM).transpose(1, 0, 2)


# speedup vs baseline: 3.7420x; 1.0010x over previous
import functools

import jax
import jax.numpy as jnp
from jax import lax
from jax.experimental import pallas as pl
from jax.experimental.pallas import tpu as pltpu
from jax.experimental.pallas import tpu_sc as plsc

VOCAB_DIM = 768
B_TOK = 1024
L_TOK = 50
N_ROWS = B_TOK * L_TOK

_info = plsc.get_sparse_core_info()
NC = _info.num_cores
NS = _info.num_subcores
NW = NC * NS
B_PER_W = N_ROWS // NW

CHUNK = 32
N_CHUNKS = B_PER_W // CHUNK
NBUF = 5


def _sc_gather(toks_grouped, table):
    mesh = plsc.VectorSubcoreMesh(core_axis_name="c", subcore_axis_name="s")

    @functools.partial(
        pl.kernel,
        mesh=mesh,
        out_type=jax.ShapeDtypeStruct((N_ROWS, VOCAB_DIM), jnp.float32),
        scratch_types=[
            pltpu.VMEM((N_CHUNKS, CHUNK), jnp.int32),
        ]
        + [pltpu.VMEM((CHUNK, VOCAB_DIM), jnp.float32) for _ in range(NBUF)]
        + [pltpu.SemaphoreType.DMA for _ in range(2 * NBUF)],
    )
    def k(toks_hbm, table_hbm, out_hbm, idx_v, *scratch):
        bufs = scratch[:NBUF]
        gsem = scratch[NBUF : 2 * NBUF]
        wsem = scratch[2 * NBUF :]
        wid = lax.axis_index("s") * NC + lax.axis_index("c")
        base = wid * B_PER_W
        pltpu.sync_copy(toks_hbm.at[wid], idx_v)

        def gather(j):
            q = j % NBUF
            return pltpu.async_copy(table_hbm.at[idx_v.at[j]], bufs[q], gsem[q])

        def writeback(c):
            p = c % NBUF
            return pltpu.async_copy(
                bufs[p], out_hbm.at[pl.ds(base + c * CHUNK, CHUNK)], wsem[p]
            )

        gh = {}
        wh = {}
        gh[0] = gather(0)
        gh[1] = gather(1)
        for c in range(N_CHUNKS):
            j = c + 2
            if j < N_CHUNKS:
                if j >= NBUF:
                    wh[j - NBUF].wait()
                gh[j] = gather(j)
            gh[c].wait()
            wh[c] = writeback(c)
        for c in range(N_CHUNKS - NBUF, N_CHUNKS):
            wh[c].wait()

    return k(toks_grouped, table)


@jax.jit
def kernel(toks, table):
    toks_grouped = toks.T.reshape(NW, N_CHUNKS, CHUNK)
    flat = _sc_gather(toks_grouped, table)
    return flat.reshape(L_TOK, B_TOK, VOCAB_DIM).transpose(1, 0, 2)
